# baseline (speedup 1.0000x reference)
import functools

import jax
import jax.numpy as jnp
from jax import lax
from jax.experimental import pallas as pl
from jax.experimental.pallas import tpu as pltpu
from jax.experimental.pallas import tpu_sc as plsc

N_NODES = 10000
N_PAD = 10240
D = 128
DH = D // 2
NC = 2
NS = 16
CH = 128
ROWS_PER_TILE = N_PAD // NS
WB = 128
DEG_W = 16


def _sc_agg_body(args, *, n_chunks, compute_deg):
    if compute_deg:
        (xlo_hbm, xhi_hbm, src3_hbm, dst3_hbm,
         agglo_hbm, agghi_hbm, deg0_hbm, deg1_hbm,
         src3_v, dst3_v, rows0, rows1, ones_v, wb_v, zd_v, dumi_v,
         agg_sh, deg_sh, sem_g0, sem_g1, sem_s0, sem_s1, sem_d) = args
    else:
        (xlo_hbm, xhi_hbm, src3_hbm, dst3_hbm,
         agglo_hbm, agghi_hbm,
         src3_v, dst3_v, rows0, rows1, wb_v,
         agg_sh, sem_g0, sem_g1, sem_s0, sem_s1) = args

    c = lax.axis_index("c")
    s = lax.axis_index("s")
    r0 = s * ROWS_PER_TILE

    zeros16 = jnp.zeros((16,), jnp.float32)
    ones16 = jnp.ones((16,), jnp.float32)

    def _fill_wb(i, carry):
        for j in range(DH // 16):
            wb_v[i, pl.ds(j * 16, 16)] = zeros16
        return carry
    lax.fori_loop(0, WB, _fill_wb, 0)

    if compute_deg:
        def _fill_zd(i, carry):
            zd_v[i, pl.ds(0, 16)] = zeros16
            return carry
        lax.fori_loop(0, ROWS_PER_TILE, _fill_zd, 0)

        def _fill_ones(i, carry):
            ones_v[i, pl.ds(0, 16)] = ones16
            return carry
        lax.fori_loop(0, CH, _fill_ones, 0)

        pad16 = jnp.full((16,), N_NODES, jnp.int32)

        def _fill_dumi(i, carry):
            dumi_v[pl.ds(i * 16, 16)] = pad16
            return carry
        lax.fori_loop(0, CH // 16, _fill_dumi, 0)

    for k in range(ROWS_PER_TILE // WB):
        pltpu.sync_copy(wb_v, agg_sh.at[pl.ds(r0 + k * WB, WB)])
    if compute_deg:
        pltpu.sync_copy(zd_v, deg_sh.at[pl.ds(r0, ROWS_PER_TILE)])

    pltpu.sync_copy(src3_hbm.at[s], src3_v)
    pltpu.sync_copy(dst3_hbm.at[s], dst3_v)
    plsc.subcore_barrier()

    def _pipeline(x_ref, deg_on_even):
        def _gather(t, rows, sem):
            pltpu.async_copy(x_ref.at[src3_v.at[t]], rows, sem)

        def _gather_wait(t, rows, sem):
            pltpu.make_async_copy(
                x_ref.at[src3_v.at[t]], rows, sem).wait()

        def _scatter(t, rows, sem):
            pltpu.async_copy(rows, agg_sh.at[dst3_v.at[t]], sem, add=True)

        def _scatter_wait(t, rows, sem):
            pltpu.make_async_copy(
                rows, agg_sh.at[dst3_v.at[t]], sem).wait()

        def _deg(t):
            pltpu.make_async_copy(
                ones_v, deg_sh.at[dst3_v.at[0]], sem_d).wait()
            pltpu.async_copy(ones_v, deg_sh.at[dst3_v.at[t]], sem_d,
                             add=True)

        if compute_deg:
            pltpu.async_copy(ones_v, deg_sh.at[dumi_v], sem_d, add=True)
        _gather(0, rows0, sem_g0)
        _gather(1, rows1, sem_g1)

        def _pair(p, carry):
            t0 = 2 * p
            t1 = t0 + 1
            _gather_wait(t0, rows0, sem_g0)
            _scatter(t0, rows0, sem_s0)
            if compute_deg and deg_on_even:
                _deg(t0)
            _gather_wait(t1, rows1, sem_g1)
            _scatter(t1, rows1, sem_s1)
            if compute_deg and not deg_on_even:
                _deg(t1)
            _scatter_wait(t0, rows0, sem_s0)
            _gather(t0 + 2, rows0, sem_g0)
            _scatter_wait(t1, rows1, sem_s1)
            _gather(t1 + 2, rows1, sem_g1)
            return carry
        lax.fori_loop(0, n_chunks // 2 - 1, _pair, 0)

        t0 = n_chunks - 2
        t1 = n_chunks - 1
        _gather_wait(t0, rows0, sem_g0)
        _scatter(t0, rows0, sem_s0)
        if compute_deg and deg_on_even:
            _deg(t0)
        _gather_wait(t1, rows1, sem_g1)
        _scatter(t1, rows1, sem_s1)
        if compute_deg and not deg_on_even:
            _deg(t1)
        _scatter_wait(t0, rows0, sem_s0)
        _scatter_wait(t1, rows1, sem_s1)
        if compute_deg:
            pltpu.make_async_copy(
                ones_v, deg_sh.at[dst3_v.at[0]], sem_d).wait()

    @pl.when(c == 0)
    def _():
        _pipeline(xlo_hbm, True)

    @pl.when(c == 1)
    def _():
        _pipeline(xhi_hbm, False)

    plsc.subcore_barrier()

    def _writeback(agg_out, deg_out):
        for k in range(ROWS_PER_TILE // WB):
            pltpu.sync_copy(agg_sh.at[pl.ds(r0 + k * WB, WB)], wb_v)
            pltpu.sync_copy(wb_v, agg_out.at[pl.ds(r0 + k * WB, WB)])
        if compute_deg:
            pltpu.sync_copy(deg_sh.at[pl.ds(r0, ROWS_PER_TILE)], zd_v)
            pltpu.sync_copy(zd_v, deg_out.at[pl.ds(r0, ROWS_PER_TILE)])

    @pl.when(c == 0)
    def _():
        _writeback(agglo_hbm, deg0_hbm if compute_deg else None)

    @pl.when(c == 1)
    def _():
        _writeback(agghi_hbm, deg1_hbm if compute_deg else None)


@functools.lru_cache(maxsize=None)
def _make_sc_agg(n_chunks, compute_deg):
    mesh = plsc.VectorSubcoreMesh(core_axis_name="c", subcore_axis_name="s")

    def body(*args):
        _sc_agg_body(args, n_chunks=n_chunks, compute_deg=compute_deg)

    out_type = [
        jax.ShapeDtypeStruct((N_PAD, DH), jnp.float32),
        jax.ShapeDtypeStruct((N_PAD, DH), jnp.float32),
    ]
    scratch = [
        pltpu.VMEM((n_chunks, CH), jnp.int32),
        pltpu.VMEM((n_chunks, CH), jnp.int32),
        pltpu.VMEM((CH, DH), jnp.float32),
        pltpu.VMEM((CH, DH), jnp.float32),
    ]
    if compute_deg:
        out_type += [
            jax.ShapeDtypeStruct((N_PAD, DEG_W), jnp.float32),
            jax.ShapeDtypeStruct((N_PAD, DEG_W), jnp.float32),
        ]
        scratch.append(pltpu.VMEM((CH, DEG_W), jnp.float32))
    scratch.append(pltpu.VMEM((WB, DH), jnp.float32))
    if compute_deg:
        scratch.append(pltpu.VMEM((ROWS_PER_TILE, DEG_W), jnp.float32))
        scratch.append(pltpu.VMEM((CH,), jnp.int32))
    scratch.append(pltpu.VMEM_SHARED((N_PAD, DH), jnp.float32))
    if compute_deg:
        scratch.append(pltpu.VMEM_SHARED((N_PAD, DEG_W), jnp.float32))
    scratch += [pltpu.SemaphoreType.DMA] * (5 if compute_deg else 4)

    return pl.kernel(
        body,
        out_type=tuple(out_type),
        mesh=mesh,
        scratch_types=scratch,
        compiler_params=pltpu.CompilerParams(use_tc_tiling_on_sc=False),
    )


def _combine0_body(agglo, agghi, deg0, deg1, x, wl, b, wr, olo, ohi):
    a = jnp.concatenate([agglo[...], agghi[...]], axis=1)
    deg = deg0[:, 0] + deg1[:, 0]
    inv = 1.0 / jnp.clip(deg, 1.0, None)
    a = a * inv[:, None]
    y = lax.dot_general(a, wl[...], (((1,), (1,)), ((), ())),
                        preferred_element_type=jnp.float32)
    y = y + lax.dot_general(x[...], wr[...], (((1,), (1,)), ((), ())),
                            preferred_element_type=jnp.float32)
    h = jnp.maximum(y + b[...], 0.0)
    olo[...] = h[:, :DH]
    ohi[...] = h[:, DH:]


def _combine0(agglo, agghi, deg0, deg1, x, W_l, b_l, W_r):
    n = x.shape[0]
    r = 1000
    return pl.pallas_call(
        _combine0_body,
        grid=(n // r,),
        in_specs=[
            pl.BlockSpec((r, DH), lambda i: (i, 0)),
            pl.BlockSpec((r, DH), lambda i: (i, 0)),
            pl.BlockSpec((r, DEG_W), lambda i: (i, 0)),
            pl.BlockSpec((r, DEG_W), lambda i: (i, 0)),
            pl.BlockSpec((r, D), lambda i: (i, 0)),
            pl.BlockSpec((D, D), lambda i: (0, 0)),
            pl.BlockSpec((1, D), lambda i: (0, 0)),
            pl.BlockSpec((D, D), lambda i: (0, 0)),
        ],
        out_specs=[
            pl.BlockSpec((r, DH), lambda i: (i, 0)),
            pl.BlockSpec((r, DH), lambda i: (i, 0)),
        ],
        out_shape=[
            jax.ShapeDtypeStruct((n, DH), jnp.float32),
            jax.ShapeDtypeStruct((n, DH), jnp.float32),
        ],
    )(agglo, agghi, deg0, deg1, x, W_l, b_l, W_r)


def _combine1_body(agglo, agghi, deg0, deg1, xlo, xhi, wl, b, wrlo, wrhi, o):
    a = jnp.concatenate([agglo[...], agghi[...]], axis=1)
    deg = deg0[:, 0] + deg1[:, 0]
    inv = 1.0 / jnp.clip(deg, 1.0, None)
    a = a * inv[:, None]
    y = lax.dot_general(a, wl[...], (((1,), (1,)), ((), ())),
                        preferred_element_type=jnp.float32)
    y = y + lax.dot_general(xlo[...], wrlo[...], (((1,), (1,)), ((), ())),
                            preferred_element_type=jnp.float32)
    y = y + lax.dot_general(xhi[...], wrhi[...], (((1,), (1,)), ((), ())),
                            preferred_element_type=jnp.float32)
    o[...] = jnp.maximum(y + b[...], 0.0)


def _combine1(agglo, agghi, deg0, deg1, xlo, xhi, W_l, b_l, W_r):
    n = xlo.shape[0]
    r = 1000
    return pl.pallas_call(
        _combine1_body,
        grid=(n // r,),
        in_specs=[
            pl.BlockSpec((r, DH), lambda i: (i, 0)),
            pl.BlockSpec((r, DH), lambda i: (i, 0)),
            pl.BlockSpec((r, DEG_W), lambda i: (i, 0)),
            pl.BlockSpec((r, DEG_W), lambda i: (i, 0)),
            pl.BlockSpec((r, DH), lambda i: (i, 0)),
            pl.BlockSpec((r, DH), lambda i: (i, 0)),
            pl.BlockSpec((D, D), lambda i: (0, 0)),
            pl.BlockSpec((1, D), lambda i: (0, 0)),
            pl.BlockSpec((D, DH), lambda i: (0, 0)),
            pl.BlockSpec((D, DH), lambda i: (0, 0)),
        ],
        out_specs=pl.BlockSpec((r, D), lambda i: (i, 0)),
        out_shape=jax.ShapeDtypeStruct((n, D), jnp.float32),
    )(agglo, agghi, deg0, deg1, xlo, xhi, W_l, b_l,
      W_r[:, :DH], W_r[:, DH:])


def kernel(x, edge_index, W_l0, b_l0, W_r0, W_l1, b_l1, W_r1):
    src = edge_index[0].astype(jnp.int32)
    dst = edge_index[1].astype(jnp.int32)
    n_edges = src.shape[0]

    n_chunks = -(-n_edges // (NS * CH))
    if n_chunks % 2:
        n_chunks += 1
    e_pad = NS * n_chunks * CH - n_edges
    src3 = jnp.concatenate(
        [src, jnp.zeros((e_pad,), jnp.int32)]).reshape(NS, n_chunks, CH)
    dst3 = jnp.concatenate(
        [dst, jnp.full((e_pad,), N_NODES, jnp.int32)]).reshape(NS, n_chunks, CH)

    sc0 = _make_sc_agg(n_chunks, True)
    sc1 = _make_sc_agg(n_chunks, False)

    agglo, agghi, deg0, deg1 = sc0(x[:, :DH], x[:, DH:], src3, dst3)
    h_lo, h_hi = _combine0(agglo, agghi, deg0, deg1, x,
                           W_l0, b_l0.reshape(1, D), W_r0)
    agglo, agghi = sc1(h_lo, h_hi, src3, dst3)
    return _combine1(agglo, agghi, deg0, deg1, h_lo, h_hi,
                     W_l1, b_l1.reshape(1, D), W_r1)


# speedup vs baseline: 1.0148x; 1.0148x over previous
import functools

import jax
import jax.numpy as jnp
from jax import lax
from jax.experimental import pallas as pl
from jax.experimental.pallas import tpu as pltpu
from jax.experimental.pallas import tpu_sc as plsc

N_NODES = 10000
N_PAD = 10240
D = 128
DH = D // 2
NC = 2
NS = 16
CH = 128
ROWS_PER_TILE = N_PAD // NS
WB = 128
DEG_W = 16


def _sc_agg_body(args, *, n_chunks, compute_deg):
    if compute_deg:
        (xlo_hbm, xhi_hbm, src3_hbm, dst3_hbm,
         agglo_hbm, agghi_hbm, deg0_hbm, deg1_hbm,
         src3_v, dst3_v, rows0, rows1, ones_v, wb_v, zd_v, dumi_v,
         agg_sh, deg_sh, sem_g0, sem_g1, sem_s0, sem_s1, sem_d) = args
    else:
        (xlo_hbm, xhi_hbm, src3_hbm, dst3_hbm,
         agglo_hbm, agghi_hbm,
         src3_v, dst3_v, rows0, rows1, wb_v,
         agg_sh, sem_g0, sem_g1, sem_s0, sem_s1) = args

    c = lax.axis_index("c")
    s = lax.axis_index("s")
    r0 = s * ROWS_PER_TILE

    zeros16 = jnp.zeros((16,), jnp.float32)
    ones16 = jnp.ones((16,), jnp.float32)

    def _fill_wb(i, carry):
        for j in range(DH // 16):
            wb_v[i, pl.ds(j * 16, 16)] = zeros16
        return carry
    lax.fori_loop(0, WB, _fill_wb, 0)

    if compute_deg:
        def _fill_zd(i, carry):
            zd_v[i, pl.ds(0, 16)] = zeros16
            return carry
        lax.fori_loop(0, ROWS_PER_TILE, _fill_zd, 0)

        def _fill_ones(i, carry):
            ones_v[i, pl.ds(0, 16)] = ones16
            return carry
        lax.fori_loop(0, CH, _fill_ones, 0)

        pad16 = jnp.full((16,), N_NODES, jnp.int32)

        def _fill_dumi(i, carry):
            dumi_v[pl.ds(i * 16, 16)] = pad16
            return carry
        lax.fori_loop(0, CH // 16, _fill_dumi, 0)

    for k in range(ROWS_PER_TILE // WB):
        pltpu.sync_copy(wb_v, agg_sh.at[pl.ds(r0 + k * WB, WB)])
    if compute_deg:
        pltpu.sync_copy(zd_v, deg_sh.at[pl.ds(r0, ROWS_PER_TILE)])

    pltpu.sync_copy(src3_hbm.at[s], src3_v)
    pltpu.sync_copy(dst3_hbm.at[s], dst3_v)
    plsc.subcore_barrier()

    def _pipeline(x_ref, deg_on_even):
        def _gather(t, rows, sem):
            pltpu.async_copy(x_ref.at[src3_v.at[t]], rows, sem)

        def _gather_wait(t, rows, sem):
            pltpu.make_async_copy(
                x_ref.at[src3_v.at[t]], rows, sem).wait()

        def _scatter(t, rows, sem):
            pltpu.async_copy(rows, agg_sh.at[dst3_v.at[t]], sem, add=True)

        def _scatter_wait(t, rows, sem):
            pltpu.make_async_copy(
                rows, agg_sh.at[dst3_v.at[t]], sem).wait()

        def _deg(t):
            pltpu.make_async_copy(
                ones_v, deg_sh.at[dst3_v.at[0]], sem_d).wait()
            pltpu.async_copy(ones_v, deg_sh.at[dst3_v.at[t]], sem_d,
                             add=True)

        if compute_deg:
            pltpu.async_copy(ones_v, deg_sh.at[dumi_v], sem_d, add=True)
        _gather(0, rows0, sem_g0)
        _gather(1, rows1, sem_g1)

        def _pair(p, carry):
            t0 = 2 * p
            t1 = t0 + 1
            _gather_wait(t0, rows0, sem_g0)
            _scatter(t0, rows0, sem_s0)
            if compute_deg and deg_on_even:
                _deg(t0)
            _gather_wait(t1, rows1, sem_g1)
            _scatter(t1, rows1, sem_s1)
            if compute_deg and not deg_on_even:
                _deg(t1)
            _scatter_wait(t0, rows0, sem_s0)
            _gather(t0 + 2, rows0, sem_g0)
            _scatter_wait(t1, rows1, sem_s1)
            _gather(t1 + 2, rows1, sem_g1)
            return carry
        lax.fori_loop(0, n_chunks // 2 - 1, _pair, 0)

        t0 = n_chunks - 2
        t1 = n_chunks - 1
        _gather_wait(t0, rows0, sem_g0)
        _scatter(t0, rows0, sem_s0)
        if compute_deg and deg_on_even:
            _deg(t0)
        _gather_wait(t1, rows1, sem_g1)
        _scatter(t1, rows1, sem_s1)
        if compute_deg and not deg_on_even:
            _deg(t1)
        _scatter_wait(t0, rows0, sem_s0)
        _scatter_wait(t1, rows1, sem_s1)
        if compute_deg:
            pltpu.make_async_copy(
                ones_v, deg_sh.at[dst3_v.at[0]], sem_d).wait()

    @pl.when(c == 0)
    def _():
        _pipeline(xlo_hbm, True)

    @pl.when(c == 1)
    def _():
        _pipeline(xhi_hbm, False)

    plsc.subcore_barrier()

    def _writeback(agg_out, deg_out):
        for k in range(ROWS_PER_TILE // WB):
            pltpu.sync_copy(agg_sh.at[pl.ds(r0 + k * WB, WB)], wb_v)
            pltpu.sync_copy(wb_v, agg_out.at[pl.ds(r0 + k * WB, WB)])
        if compute_deg:
            pltpu.sync_copy(deg_sh.at[pl.ds(r0, ROWS_PER_TILE)], zd_v)
            pltpu.sync_copy(zd_v, deg_out.at[pl.ds(r0, ROWS_PER_TILE)])

    @pl.when(c == 0)
    def _():
        _writeback(agglo_hbm, deg0_hbm if compute_deg else None)

    @pl.when(c == 1)
    def _():
        _writeback(agghi_hbm, deg1_hbm if compute_deg else None)


@functools.lru_cache(maxsize=None)
def _make_sc_agg(n_chunks, compute_deg):
    mesh = plsc.VectorSubcoreMesh(core_axis_name="c", subcore_axis_name="s")

    def body(*args):
        _sc_agg_body(args, n_chunks=n_chunks, compute_deg=compute_deg)

    out_type = [
        jax.ShapeDtypeStruct((N_PAD, DH), jnp.float32),
        jax.ShapeDtypeStruct((N_PAD, DH), jnp.float32),
    ]
    scratch = [
        pltpu.VMEM((n_chunks, CH), jnp.int32),
        pltpu.VMEM((n_chunks, CH), jnp.int32),
        pltpu.VMEM((CH, DH), jnp.float32),
        pltpu.VMEM((CH, DH), jnp.float32),
    ]
    if compute_deg:
        out_type += [
            jax.ShapeDtypeStruct((N_PAD, DEG_W), jnp.float32),
            jax.ShapeDtypeStruct((N_PAD, DEG_W), jnp.float32),
        ]
        scratch.append(pltpu.VMEM((CH, DEG_W), jnp.float32))
    scratch.append(pltpu.VMEM((WB, DH), jnp.float32))
    if compute_deg:
        scratch.append(pltpu.VMEM((ROWS_PER_TILE, DEG_W), jnp.float32))
        scratch.append(pltpu.VMEM((CH,), jnp.int32))
    scratch.append(pltpu.VMEM_SHARED((N_PAD, DH), jnp.float32))
    if compute_deg:
        scratch.append(pltpu.VMEM_SHARED((N_PAD, DEG_W), jnp.float32))
    scratch += [pltpu.SemaphoreType.DMA] * (5 if compute_deg else 4)

    return pl.kernel(
        body,
        out_type=tuple(out_type),
        mesh=mesh,
        scratch_types=scratch,
        compiler_params=pltpu.CompilerParams(use_tc_tiling_on_sc=False),
    )


def _combine_body(agglo, agghi, deg0, deg1, x, wl, b, wr, o):
    a = jnp.concatenate([agglo[...], agghi[...]], axis=1)
    deg = deg0[:, 0] + deg1[:, 0]
    inv = 1.0 / jnp.clip(deg, 1.0, None)
    a = a * inv[:, None]
    y = lax.dot_general(a, wl[...], (((1,), (1,)), ((), ())),
                        preferred_element_type=jnp.float32)
    y = y + lax.dot_general(x[...], wr[...], (((1,), (1,)), ((), ())),
                            preferred_element_type=jnp.float32)
    o[...] = jnp.maximum(y + b[...], 0.0)


def _combine(agglo, agghi, deg0, deg1, x, W_l, b_l, W_r):
    n = x.shape[0]
    r = 1000
    grid = (n // r,)
    return pl.pallas_call(
        _combine_body,
        grid=grid,
        in_specs=[
            pl.BlockSpec((r, DH), lambda i: (i, 0)),
            pl.BlockSpec((r, DH), lambda i: (i, 0)),
            pl.BlockSpec((r, DEG_W), lambda i: (i, 0)),
            pl.BlockSpec((r, DEG_W), lambda i: (i, 0)),
            pl.BlockSpec((r, D), lambda i: (i, 0)),
            pl.BlockSpec((D, D), lambda i: (0, 0)),
            pl.BlockSpec((1, D), lambda i: (0, 0)),
            pl.BlockSpec((D, D), lambda i: (0, 0)),
        ],
        out_specs=pl.BlockSpec((r, D), lambda i: (i, 0)),
        out_shape=jax.ShapeDtypeStruct((n, D), jnp.float32),
    )(agglo, agghi, deg0, deg1, x, W_l, b_l, W_r)


def kernel(x, edge_index, W_l0, b_l0, W_r0, W_l1, b_l1, W_r1):
    src = edge_index[0].astype(jnp.int32)
    dst = edge_index[1].astype(jnp.int32)
    n_edges = src.shape[0]

    n_chunks = -(-n_edges // (NS * CH))
    if n_chunks % 2:
        n_chunks += 1
    e_pad = NS * n_chunks * CH - n_edges
    src3 = jnp.concatenate(
        [src, jnp.zeros((e_pad,), jnp.int32)]).reshape(NS, n_chunks, CH)
    dst3 = jnp.concatenate(
        [dst, jnp.full((e_pad,), N_NODES, jnp.int32)]).reshape(NS, n_chunks, CH)

    sc0 = _make_sc_agg(n_chunks, True)
    sc1 = _make_sc_agg(n_chunks, False)

    agglo, agghi, deg0, deg1 = sc0(x[:, :DH], x[:, DH:], src3, dst3)
    h = _combine(agglo, agghi, deg0, deg1, x, W_l0, b_l0.reshape(1, D), W_r0)
    agglo, agghi = sc1(h[:, :DH], h[:, DH:], src3, dst3)
    return _combine(agglo, agghi, deg0, deg1, h, W_l1, b_l1.reshape(1, D), W_r1)
